# baseline (speedup 1.0000x reference)
import jax
import jax.numpy as jnp
from jax import lax
from jax.experimental import pallas as pl
from jax.experimental.pallas import tpu as pltpu
from jax.experimental.pallas import tpu_sc as plsc

_N = 10000
_E = 320000
_D = 128
_T = 4
_STEPS = 8

_DP = _D // 2
_NC, _NS = 2, 16
_NW = _NC * _NS
_EPT = _E // _NW
_CH = 40
_NCH = _EPT // _CH
_NP = 10240
_RPT = _NP // _NS
_ZR = 40
_NZ = _RPT // _ZR


def _sc_body(table, gidx, dstv, part, acc, gidx_v, db0, db1, db2, db3,
             rr0, rr1, rr2, rr3, sg0, sg1, sg2, sg3, sd0, sd1, sd2, sd3,
             ss0, ss1, ss2, ss3, sem_z):
    c = lax.axis_index("c")
    s = lax.axis_index("s")
    wid = s * _NC + c
    pltpu.sync_copy(gidx.at[wid], gidx_v)

    dbs = (db0, db1, db2, db3)
    sds = (sd0, sd1, sd2, sd3)
    rrs = (rr0, rr1, rr2, rr3)
    sgs = (sg0, sg1, sg2, sg3)
    sss = (ss0, ss1, ss2, ss3)

    def _gfire(j, rr, sem):
        pltpu.async_copy(table.at[gidx_v.at[pl.ds(j * _CH, _CH)]], rr, sem)

    def _gwait(j, rr, sem):
        pltpu.make_async_copy(table.at[gidx_v.at[pl.ds(j * _CH, _CH)]],
                              rr, sem).wait()

    def _dfire(j, db, sem):
        pltpu.async_copy(dstv.at[wid].at[j], db, sem)

    def _dwait(j, db, sem):
        pltpu.make_async_copy(dstv.at[wid].at[j], db, sem).wait()

    def _sfire(rr, db, sem):
        pltpu.async_copy(rr, acc.at[db.at[0]], sem, add=True)

    def _swait(rr, db, sem):
        pltpu.make_async_copy(rr, acc.at[db.at[0]], sem).wait()

    _gfire(0, rr0, sg0)
    _gfire(1, rr1, sg1)
    _dfire(0, db0, sd0)
    _dfire(1, db1, sd1)
    zv = jnp.zeros((16,), jnp.float32)

    def _zrow(i, _):
        for k in range(_D // 16):
            rr2[i, pl.ds(k * 16, 16)] = zv
        return 0

    lax.fori_loop(0, _ZR, _zrow, 0)
    for i in range(_NZ):
        pltpu.async_copy(rr2, acc.at[pl.ds(s * _RPT + i * _ZR, _ZR)], sem_z)
    for i in range(_NZ):
        pltpu.make_async_copy(
            rr2, acc.at[pl.ds(s * _RPT + i * _ZR, _ZR)], sem_z).wait()
    plsc.subcore_barrier()

    def _unit(ci, u, in_loop):
        v = (u + 2) % 4
        _gwait(ci, rrs[u], sgs[u])
        if in_loop:
            @pl.when(ci >= 2)
            def _():
                _swait(rrs[v], dbs[v], sss[v])
            _gfire(ci + 2, rrs[v], sgs[v])
            _dfire(ci + 2, dbs[v], sds[v])
        else:
            _swait(rrs[v], dbs[v], sss[v])
        _dwait(ci, dbs[u], sds[u])
        _sfire(rrs[u], dbs[u], sss[u])

    def _quad(q, _):
        for u in range(4):
            _unit(4 * q + u, u, True)
        return 0

    lax.fori_loop(0, (_NCH - 2) // 4, _quad, 0)
    _unit(_NCH - 2, 0, False)
    _unit(_NCH - 1, 1, False)
    _swait(rr0, db0, ss0)
    _swait(rr1, db1, ss1)
    plsc.subcore_barrier()

    for i in range(_NZ):
        sl = pl.ds(s * _RPT + i * _ZR, _ZR)
        buf = rr0 if i % 2 == 0 else rr1
        if i >= 2:
            pltpu.make_async_copy(
                buf, part.at[c].at[pl.ds(s * _RPT + (i - 2) * _ZR, _ZR)],
                ss0 if i % 2 == 0 else ss1).wait()
        pltpu.sync_copy(acc.at[sl], buf)
        pltpu.async_copy(buf, part.at[c].at[sl], ss0 if i % 2 == 0 else ss1)
    for i in range(_NZ - 2, _NZ):
        sl = pl.ds(s * _RPT + i * _ZR, _ZR)
        buf = rr0 if i % 2 == 0 else rr1
        pltpu.make_async_copy(
            buf, part.at[c].at[sl], ss0 if i % 2 == 0 else ss1).wait()


_sc_cache = {}


def _get_sc_aggregate():
    if "k" not in _sc_cache:
        _sc_cache["k"] = pl.kernel(
            _sc_body,
            out_type=jax.ShapeDtypeStruct((_NC, _NP, _D), jnp.float32),
            mesh=plsc.VectorSubcoreMesh(
                core_axis_name="c", subcore_axis_name="s",
                num_cores=_NC, num_subcores=_NS,
            ),
            scratch_types=[
                pltpu.VMEM_SHARED((_NP, _D), jnp.float32),
                pltpu.VMEM((_EPT,), jnp.int32),
                pltpu.VMEM((1, _CH), jnp.int32),
                pltpu.VMEM((1, _CH), jnp.int32),
                pltpu.VMEM((1, _CH), jnp.int32),
                pltpu.VMEM((1, _CH), jnp.int32),
                pltpu.VMEM((_CH, _D), jnp.float32),
                pltpu.VMEM((_CH, _D), jnp.float32),
                pltpu.VMEM((_CH, _D), jnp.float32),
                pltpu.VMEM((_CH, _D), jnp.float32),
            ] + [pltpu.SemaphoreType.DMA] * 13,
        )
    return _sc_cache["k"]


_BLK = 1000


def _init_body(h_ref, wcat_ref, bcat_ref, aall_ref):
    av = jnp.dot(h_ref[...], wcat_ref[...], preferred_element_type=jnp.float32)
    av = av + bcat_ref[...]
    for t in range(_T):
        aall_ref[t] = av[:, t * _D:(t + 1) * _D]


def _gru_body(part_ref, h_ref, wih_ref, whh_ref, bih_ref, bhh_ref, wcat_ref,
              bcat_ref, hnew_ref, aall_ref):
    a = part_ref[0] + part_ref[1]
    h = h_ref[...]
    gi = jnp.dot(a, wih_ref[...], preferred_element_type=jnp.float32)
    gi = gi + bih_ref[...]
    gh = jnp.dot(h, whh_ref[...], preferred_element_type=jnp.float32)
    gh = gh + bhh_ref[...]
    r = jax.nn.sigmoid(gi[:, :_D] + gh[:, :_D])
    z = jax.nn.sigmoid(gi[:, _D:2 * _D] + gh[:, _D:2 * _D])
    n = jnp.tanh(gi[:, 2 * _D:] + r * gh[:, 2 * _D:])
    hn = (1.0 - z) * n + z * h
    hnew_ref[...] = hn
    av = jnp.dot(hn, wcat_ref[...], preferred_element_type=jnp.float32)
    av = av + bcat_ref[...]
    for t in range(_T):
        aall_ref[t] = av[:, t * _D:(t + 1) * _D]


_full = lambda i: (0, 0)

_tc_init = pl.pallas_call(
    _init_body,
    grid=(_N // _BLK,),
    in_specs=[
        pl.BlockSpec((_BLK, _D), lambda i: (i, 0)),
        pl.BlockSpec((_D, _T * _D), _full),
        pl.BlockSpec((1, _T * _D), _full),
    ],
    out_specs=pl.BlockSpec((_T, _BLK, _D), lambda i: (0, i, 0)),
    out_shape=jax.ShapeDtypeStruct((_T, _N, _D), jnp.float32),
)

_tc_gru = pl.pallas_call(
    _gru_body,
    grid=(_N // _BLK,),
    in_specs=[
        pl.BlockSpec((_NC, _BLK, _D), lambda i: (0, i, 0)),
        pl.BlockSpec((_BLK, _D), lambda i: (i, 0)),
        pl.BlockSpec((_D, 3 * _D), _full),
        pl.BlockSpec((_D, 3 * _D), _full),
        pl.BlockSpec((1, 3 * _D), _full),
        pl.BlockSpec((1, 3 * _D), _full),
        pl.BlockSpec((_D, _T * _D), _full),
        pl.BlockSpec((1, _T * _D), _full),
    ],
    out_specs=[
        pl.BlockSpec((_BLK, _D), lambda i: (i, 0)),
        pl.BlockSpec((_T, _BLK, _D), lambda i: (0, i, 0)),
    ],
    out_shape=[
        jax.ShapeDtypeStruct((_N, _D), jnp.float32),
        jax.ShapeDtypeStruct((_T, _N, _D), jnp.float32),
    ],
)


def kernel(feats, edge_index, etypes, W, b, W_ih, W_hh, b_ih, b_hh):
    src = edge_index[0].astype(jnp.int32)
    dst = edge_index[1].astype(jnp.int32)
    et = etypes.astype(jnp.int32)
    gidx = (et * _N + src).reshape(_NW, _EPT)
    dstr = dst.reshape(_NW, _NCH, 1, _CH)
    W_cat = jnp.transpose(W, (2, 0, 1)).reshape(_D, _T * _D)
    b_cat = b.reshape(1, _T * _D)
    W_ih_t = W_ih.T
    W_hh_t = W_hh.T
    b_ih2 = b_ih.reshape(1, 3 * _D)
    b_hh2 = b_hh.reshape(1, 3 * _D)

    h = feats
    aall = _tc_init(h, W_cat, b_cat)
    sc_aggregate = _get_sc_aggregate()
    for _ in range(_STEPS):
        table = aall.reshape(_T * _N, _D)
        part = sc_aggregate(table, gidx, dstr)
        h, aall = _tc_gru(part, h, W_ih_t, W_hh_t, b_ih2, b_hh2, W_cat, b_cat)
    return h
